# baseline (speedup 1.0000x reference)
import jax
import jax.numpy as jnp
from jax import lax
from jax.experimental import pallas as pl
from jax.experimental.pallas import tpu as pltpu
from jax.experimental.pallas import tpu_sc as plsc

N = 10000
D = 128
E = 320000
HALF = D // 2
N_LAYERS = 2
K = N // 2
EROWS = E // 128
EROWS_PAD = 2560
N_PAD = 10240
N_SUB = 16
RPT = N // N_SUB
RPS = EROWS_PAD // N_SUB
GRP = 4
NG = RPS // GRP
NDUMMY = 8


def _sc_body(feat_lo, feat_hi, src2d, dst2d, zagg, zdeg, onesb,
             out_lo, out_hi, outd0, outd1,
             agg_s, deg_s, idxs, idxd, rows_b, ones_v,
             gsem0, gsem1, ssem0, ssem1, dsem0, dsem1):
    c = lax.axis_index("c")
    s = lax.axis_index("s")
    sl = pl.ds(s * RPT, RPT)
    pltpu.sync_copy(zagg, agg_s.at[sl])
    pltpu.sync_copy(zdeg, deg_s.at[sl])
    pltpu.sync_copy(onesb, ones_v)
    plsc.subcore_barrier()

    gsem = (gsem0, gsem1)
    ssem = (ssem0, ssem1)
    dsem = (dsem0, dsem1)
    row_base = s * RPS

    def load_idx(p, g):
        row0 = row_base + g * GRP
        pltpu.sync_copy(src2d.at[pl.ds(row0, GRP)], idxs.at[p])
        pltpu.sync_copy(dst2d.at[pl.ds(row0, GRP)], idxd.at[p])

    def issue_gathers(p):
        for b in range(GRP):
            @pl.when(c == 0)
            def _():
                pltpu.async_copy(feat_lo.at[idxs.at[p, b]], rows_b.at[p, b],
                                 gsem[p])

            @pl.when(c == 1)
            def _():
                pltpu.async_copy(feat_hi.at[idxs.at[p, b]], rows_b.at[p, b],
                                 gsem[p])

    def wait_gathers(p):
        for b in range(GRP):
            pltpu.make_async_copy(feat_lo.at[idxs.at[p, b]], rows_b.at[p, b],
                                  gsem[p]).wait()

    def issue_scatters(p):
        for b in range(GRP):
            pltpu.async_copy(rows_b.at[p, b], agg_s.at[idxd.at[p, b]],
                             ssem[p], add=True)

            @pl.when(c == b % 2)
            def _():
                pltpu.async_copy(ones_v, deg_s.at[idxd.at[p, b]],
                                 dsem[p], add=True)

    def wait_scatters(p):
        for b in range(GRP):
            pltpu.make_async_copy(rows_b.at[p, b], agg_s.at[idxd.at[p, b]],
                                  ssem[p]).wait()

            @pl.when(c == b % 2)
            def _():
                pltpu.make_async_copy(ones_v, deg_s.at[idxd.at[p, b]],
                                      dsem[p]).wait()

    load_idx(0, 0)
    issue_gathers(0)

    @pl.loop(0, NG, step=2)
    def _grp(gg):
        load_idx(1, gg + 1)
        wait_gathers(0)
        issue_scatters(0)
        issue_gathers(1)
        wait_scatters(0)

        @pl.when(gg < NG - 2)
        def _():
            load_idx(0, gg + 2)

        wait_gathers(1)
        issue_scatters(1)

        @pl.when(gg < NG - 2)
        def _():
            issue_gathers(0)

        wait_scatters(1)

    plsc.subcore_barrier()

    @pl.when(c == 0)
    def _():
        pltpu.sync_copy(agg_s.at[sl], out_lo.at[sl])
        pltpu.sync_copy(deg_s.at[sl], outd0.at[sl])

    @pl.when(c == 1)
    def _():
        pltpu.sync_copy(agg_s.at[sl], out_hi.at[sl])
        pltpu.sync_copy(deg_s.at[sl], outd1.at[sl])


def _sc_aggregate(feat_lo, feat_hi, src2d, dst2d):
    f32 = jnp.float32
    zagg = jnp.zeros((RPT, HALF), f32)
    zdeg = jnp.zeros((RPT, 16), f32)
    onesb = jnp.zeros((128, 16), f32).at[:, 0].set(1.0)
    run = pl.kernel(
        _sc_body,
        out_type=(
            jax.ShapeDtypeStruct((N, HALF), f32),
            jax.ShapeDtypeStruct((N, HALF), f32),
            jax.ShapeDtypeStruct((N, 16), f32),
            jax.ShapeDtypeStruct((N, 16), f32),
        ),
        mesh=plsc.VectorSubcoreMesh(core_axis_name="c", subcore_axis_name="s"),
        compiler_params=pltpu.CompilerParams(use_tc_tiling_on_sc=False),
        scratch_types=(
            pltpu.VMEM_SHARED((N + NDUMMY, HALF), f32),
            pltpu.VMEM_SHARED((N + NDUMMY, 16), f32),
            pltpu.VMEM((2, GRP, 128), jnp.int32),
            pltpu.VMEM((2, GRP, 128), jnp.int32),
            pltpu.VMEM((2, GRP, 128, HALF), f32),
            pltpu.VMEM((128, 16), f32),
            pltpu.SemaphoreType.DMA,
            pltpu.SemaphoreType.DMA,
            pltpu.SemaphoreType.DMA,
            pltpu.SemaphoreType.DMA,
            pltpu.SemaphoreType.DMA,
            pltpu.SemaphoreType.DMA,
        ),
    )
    return run(feat_lo, feat_hi, src2d, dst2d, zagg, zdeg, onesb)


def _sim_body(slo, shi, d0, d1, ori, sim):
    deg = d0[:, 0:1] + d1[:, 0:1]
    degc = jnp.maximum(deg, 1.0)
    agg_lo = slo[...] / degc
    agg_hi = shi[...] / degc
    o = ori[...]
    o_lo = o[:, :HALF]
    o_hi = o[:, HALF:]
    num = (jnp.sum(agg_lo * o_lo, axis=1, keepdims=True)
           + jnp.sum(agg_hi * o_hi, axis=1, keepdims=True))
    ssq = (jnp.sum(agg_lo * agg_lo, axis=1, keepdims=True)
           + jnp.sum(agg_hi * agg_hi, axis=1, keepdims=True))
    osq = jnp.sum(o * o, axis=1, keepdims=True)
    denom = jnp.sqrt(ssq + 1e-12) * jnp.sqrt(osq + 1e-12) + 1e-8
    sim[...] = (num / denom).reshape(1, 1, 128)


def _tc_sims(sum_lo, sum_hi, deg0, deg1, ori):
    bm = 128
    grid = (N + bm - 1) // bm
    return pl.pallas_call(
        _sim_body,
        grid=(grid,),
        in_specs=[
            pl.BlockSpec((bm, HALF), lambda i: (i, 0)),
            pl.BlockSpec((bm, HALF), lambda i: (i, 0)),
            pl.BlockSpec((bm, 16), lambda i: (i, 0)),
            pl.BlockSpec((bm, 16), lambda i: (i, 0)),
            pl.BlockSpec((bm, D), lambda i: (i, 0)),
        ],
        out_specs=pl.BlockSpec((1, 1, 128), lambda i: (i, 0, 0)),
        out_shape=jax.ShapeDtypeStruct((N_PAD // 128, 1, 128), jnp.float32),
    )(sum_lo, sum_hi, deg0, deg1, ori)


def _u_of(sim, idx):
    b = lax.bitcast_convert_type(sim, jnp.int32)
    key = jnp.where(b < 0, b ^ jnp.int32(0x7FFFFFFF), b)
    u = lax.bitcast_convert_type(key, jnp.uint32) ^ jnp.uint32(0x80000000)
    return jnp.where(idx < N, u, jnp.uint32(0))


def _msel_body(sim80, slo, shi, d0, d1, xlo, xhi, olo, ohi, smem):
    i = pl.program_id(0)

    @pl.when(i == 0)
    def _():
        sim = sim80[...]
        idx = (lax.broadcasted_iota(jnp.int32, sim.shape, 0) * 128
               + lax.broadcasted_iota(jnp.int32, sim.shape, 1))
        u = _u_of(sim, idx)

        def bit_step(j, prefix):
            bit = lax.shift_left(jnp.uint32(1),
                                 jnp.uint32(31) - j.astype(jnp.uint32))
            cand = prefix | bit
            cnt = jnp.sum((u >= cand).astype(jnp.int32))
            return jnp.where(cnt >= K, cand, prefix)

        thr = lax.fori_loop(0, 32, bit_step, jnp.uint32(0))
        cnt_gt = jnp.sum((u > thr).astype(jnp.int32))
        need = K - cnt_gt
        tie = u == thr

        def m_step(j, m):
            bit = lax.shift_left(jnp.int32(1), jnp.int32(13) - j)
            cand = m | bit
            cnt = jnp.sum((tie & (idx < cand)).astype(jnp.int32))
            return jnp.where(cnt < need, cand, m)

        last = lax.fori_loop(0, 14, m_step, jnp.int32(0))
        smem[0] = lax.bitcast_convert_type(thr, jnp.int32)
        smem[1] = last

    thr = lax.bitcast_convert_type(smem[0], jnp.uint32)
    last = smem[1]
    row = sim80[pl.ds(i, 1), :]
    idx_b = i * 128 + lax.broadcasted_iota(jnp.int32, (1, 128), 1)
    ub = _u_of(row, idx_b)
    keep_row = (ub > thr) | ((ub == thr) & (idx_b <= last))
    keep = keep_row.astype(jnp.float32).reshape(128, 1) > 0.0
    deg = d0[:, 0:1] + d1[:, 0:1]
    degc = jnp.maximum(deg, 1.0)
    olo[...] = jnp.where(keep, slo[...] / degc, xlo[...])
    ohi[...] = jnp.where(keep, shi[...] / degc, xhi[...])


def _tc_mask_select(sim80, sum_lo, sum_hi, deg0, deg1, x_lo, x_hi):
    bm = 128
    grid = (N + bm - 1) // bm
    bs_h = pl.BlockSpec((bm, HALF), lambda i: (i, 0))
    bs_d = pl.BlockSpec((bm, 16), lambda i: (i, 0))
    bs_s = pl.BlockSpec((N_PAD // 128, 128), lambda i: (0, 0))
    return pl.pallas_call(
        _msel_body,
        grid=(grid,),
        in_specs=[bs_s, bs_h, bs_h, bs_d, bs_d, bs_h, bs_h],
        out_specs=(bs_h, bs_h),
        out_shape=(
            jax.ShapeDtypeStruct((N, HALF), jnp.float32),
            jax.ShapeDtypeStruct((N, HALF), jnp.float32),
        ),
        scratch_shapes=[pltpu.SMEM((2,), jnp.int32)],
    )(sim80, sum_lo, sum_hi, deg0, deg1, x_lo, x_hi)


def _layer(x_lo, x_hi, ori, src2d, dst2d):
    sum_lo, sum_hi, deg0, deg1 = _sc_aggregate(x_lo, x_hi, src2d, dst2d)
    sim80 = _tc_sims(sum_lo, sum_hi, deg0, deg1, ori).reshape(N_PAD // 128, 128)
    return _tc_mask_select(sim80, sum_lo, sum_hi, deg0, deg1, x_lo, x_hi)


def kernel(features, adj_lst):
    ori = features
    x_lo = features[:, :HALF]
    x_hi = features[:, HALF:]
    pad_rows = EROWS_PAD - EROWS
    src_pad = jnp.zeros((pad_rows, 128), jnp.int32)
    dst_pad = jnp.full((pad_rows, 128), N, jnp.int32)
    for i in range(N_LAYERS):
        src2d = jnp.concatenate(
            [adj_lst[i, 0].reshape(EROWS, 128), src_pad])
        dst2d = jnp.concatenate(
            [adj_lst[i, 1].reshape(EROWS, 128), dst_pad])
        x_lo, x_hi = _layer(x_lo, x_hi, ori, src2d, dst2d)
    return jnp.concatenate([x_lo, x_hi], axis=1)


# speedup vs baseline: 2.0758x; 2.0758x over previous
import jax
import jax.numpy as jnp
from jax import lax
from jax.experimental import pallas as pl
from jax.experimental.pallas import tpu as pltpu
from jax.experimental.pallas import tpu_sc as plsc

N = 10000
D = 128
E = 320000
HALF = D // 2
N_LAYERS = 2
K = N // 2
EROWS = E // 128
EROWS_PAD = 2560
N_PAD = 10240
N_SUB = 16
RPT = N // N_SUB
RPS = EROWS_PAD // N_SUB
GRP = 4
NG = RPS // GRP
NDUMMY = 8


def _sc_body(feat_lo, feat_hi, src2d, dst2d, zagg, zdeg, onesb,
             out_lo, out_hi, outd0, outd1,
             agg_s, deg_s, idxs, idxd, rows_b, ones_v,
             gsem0, gsem1, ssem0, ssem1, dsem0, dsem1):
    c = lax.axis_index("c")
    s = lax.axis_index("s")
    sl = pl.ds(s * RPT, RPT)
    pltpu.sync_copy(zagg, agg_s.at[sl])
    pltpu.sync_copy(zdeg, deg_s.at[sl])
    pltpu.sync_copy(onesb, ones_v)
    plsc.subcore_barrier()

    gsem = (gsem0, gsem1)
    ssem = (ssem0, ssem1)
    dsem = (dsem0, dsem1)
    row_base = s * RPS

    def load_idx(p, g):
        row0 = row_base + g * GRP

        @pl.when(row0 < EROWS)
        def _():
            pltpu.sync_copy(src2d.at[pl.ds(row0, GRP)], idxs.at[p])
            pltpu.sync_copy(dst2d.at[pl.ds(row0, GRP)], idxd.at[p])

    def issue_gathers(p, g):
        @pl.when(row_base + g * GRP < EROWS)
        def _():
            for b in range(GRP):
                @pl.when(c == 0)
                def _():
                    pltpu.async_copy(feat_lo.at[idxs.at[p, b]],
                                     rows_b.at[p, b], gsem[p])

                @pl.when(c == 1)
                def _():
                    pltpu.async_copy(feat_hi.at[idxs.at[p, b]],
                                     rows_b.at[p, b], gsem[p])

    def wait_gathers(p, g):
        @pl.when(row_base + g * GRP < EROWS)
        def _():
            for b in range(GRP):
                pltpu.make_async_copy(feat_lo.at[idxs.at[p, b]],
                                      rows_b.at[p, b], gsem[p]).wait()

    def issue_scatters(p, g):
        @pl.when(row_base + g * GRP < EROWS)
        def _():
            for b in range(GRP):
                pltpu.async_copy(rows_b.at[p, b], agg_s.at[idxd.at[p, b]],
                                 ssem[p], add=True)

                @pl.when(c == b % 2)
                def _():
                    pltpu.async_copy(ones_v, deg_s.at[idxd.at[p, b]],
                                     dsem[p], add=True)

    def wait_scatters(p, g):
        @pl.when(row_base + g * GRP < EROWS)
        def _():
            for b in range(GRP):
                pltpu.make_async_copy(rows_b.at[p, b], agg_s.at[idxd.at[p, b]],
                                      ssem[p]).wait()

                @pl.when(c == b % 2)
                def _():
                    pltpu.make_async_copy(ones_v, deg_s.at[idxd.at[p, b]],
                                          dsem[p]).wait()

    load_idx(0, 0)
    issue_gathers(0, 0)

    @pl.loop(0, NG, step=2)
    def _grp(gg):
        load_idx(1, gg + 1)
        wait_gathers(0, gg)
        issue_scatters(0, gg)
        issue_gathers(1, gg + 1)
        wait_scatters(0, gg)

        @pl.when(gg < NG - 2)
        def _():
            load_idx(0, gg + 2)

        wait_gathers(1, gg + 1)
        issue_scatters(1, gg + 1)

        @pl.when(gg < NG - 2)
        def _():
            issue_gathers(0, gg + 2)

        wait_scatters(1, gg + 1)

    plsc.subcore_barrier()

    @pl.when(c == 0)
    def _():
        pltpu.sync_copy(agg_s.at[sl], out_lo.at[sl])
        pltpu.sync_copy(deg_s.at[sl], outd0.at[sl])

    @pl.when(c == 1)
    def _():
        pltpu.sync_copy(agg_s.at[sl], out_hi.at[sl])
        pltpu.sync_copy(deg_s.at[sl], outd1.at[sl])


def _sc_aggregate(feat_lo, feat_hi, src2d, dst2d):
    f32 = jnp.float32
    zagg = jnp.zeros((RPT, HALF), f32)
    zdeg = jnp.zeros((RPT, 16), f32)
    onesb = jnp.zeros((128, 16), f32).at[:, 0].set(1.0)
    run = pl.kernel(
        _sc_body,
        out_type=(
            jax.ShapeDtypeStruct((N, HALF), f32),
            jax.ShapeDtypeStruct((N, HALF), f32),
            jax.ShapeDtypeStruct((N, 16), f32),
            jax.ShapeDtypeStruct((N, 16), f32),
        ),
        mesh=plsc.VectorSubcoreMesh(core_axis_name="c", subcore_axis_name="s"),
        compiler_params=pltpu.CompilerParams(use_tc_tiling_on_sc=False),
        scratch_types=(
            pltpu.VMEM_SHARED((N + NDUMMY, HALF), f32),
            pltpu.VMEM_SHARED((N + NDUMMY, 16), f32),
            pltpu.VMEM((2, GRP, 128), jnp.int32),
            pltpu.VMEM((2, GRP, 128), jnp.int32),
            pltpu.VMEM((2, GRP, 128, HALF), f32),
            pltpu.VMEM((128, 16), f32),
            pltpu.SemaphoreType.DMA,
            pltpu.SemaphoreType.DMA,
            pltpu.SemaphoreType.DMA,
            pltpu.SemaphoreType.DMA,
            pltpu.SemaphoreType.DMA,
            pltpu.SemaphoreType.DMA,
        ),
    )
    return run(feat_lo, feat_hi, src2d, dst2d, zagg, zdeg, onesb)


def _sim_body(slo, shi, d0, d1, ori, sim):
    deg = d0[:, 0:1] + d1[:, 0:1]
    degc = jnp.maximum(deg, 1.0)
    agg_lo = slo[...] / degc
    agg_hi = shi[...] / degc
    o = ori[...]
    o_lo = o[:, :HALF]
    o_hi = o[:, HALF:]
    num = (jnp.sum(agg_lo * o_lo, axis=1, keepdims=True)
           + jnp.sum(agg_hi * o_hi, axis=1, keepdims=True))
    ssq = (jnp.sum(agg_lo * agg_lo, axis=1, keepdims=True)
           + jnp.sum(agg_hi * agg_hi, axis=1, keepdims=True))
    osq = jnp.sum(o * o, axis=1, keepdims=True)
    denom = jnp.sqrt(ssq + 1e-12) * jnp.sqrt(osq + 1e-12) + 1e-8
    sim[...] = num / denom


def _tc_sims(sum_lo, sum_hi, deg0, deg1, ori):
    bm = 1000
    grid = N // bm
    return pl.pallas_call(
        _sim_body,
        grid=(grid,),
        in_specs=[
            pl.BlockSpec((bm, HALF), lambda i: (i, 0)),
            pl.BlockSpec((bm, HALF), lambda i: (i, 0)),
            pl.BlockSpec((bm, 16), lambda i: (i, 0)),
            pl.BlockSpec((bm, 16), lambda i: (i, 0)),
            pl.BlockSpec((bm, D), lambda i: (i, 0)),
        ],
        out_specs=pl.BlockSpec((bm, 1), lambda i: (i, 0)),
        out_shape=jax.ShapeDtypeStruct((N, 1), jnp.float32),
    )(sum_lo, sum_hi, deg0, deg1, ori)


def _mask_body(sim80, mask80):
    sim = sim80[...]
    b = lax.bitcast_convert_type(sim, jnp.int32)
    key = jnp.where(b < 0, b ^ jnp.int32(0x7FFFFFFF), b)
    u = lax.bitcast_convert_type(key, jnp.uint32) ^ jnp.uint32(0x80000000)

    def bit_step(i, prefix):
        bit = lax.shift_left(jnp.uint32(1),
                             jnp.uint32(31) - i.astype(jnp.uint32))
        cand = prefix | bit
        cnt = jnp.sum((u >= cand).astype(jnp.int32))
        return jnp.where(cnt >= K, cand, prefix)

    thr = lax.fori_loop(0, 32, bit_step, jnp.uint32(0))
    cnt_gt = jnp.sum((u > thr).astype(jnp.int32))
    need = K - cnt_gt
    tie = u == thr
    idx = (lax.broadcasted_iota(jnp.int32, sim.shape, 0) * 128
           + lax.broadcasted_iota(jnp.int32, sim.shape, 1))

    def m_step(i, m):
        bit = lax.shift_left(jnp.int32(1), jnp.int32(13) - i)
        cand = m | bit
        cnt = jnp.sum((tie & (idx < cand)).astype(jnp.int32))
        return jnp.where(cnt < need, cand, m)

    last = lax.fori_loop(0, 14, m_step, jnp.int32(0))
    keep = (u > thr) | (tie & (idx <= last))
    mask80[...] = keep.astype(jnp.float32)


def _tc_mask(sim80):
    return pl.pallas_call(
        _mask_body,
        out_shape=jax.ShapeDtypeStruct((N_PAD // 128, 128), jnp.float32),
    )(sim80)


def _sel_body(slo, shi, d0, d1, m, xlo, xhi, olo, ohi):
    deg = d0[:, 0:1] + d1[:, 0:1]
    degc = jnp.maximum(deg, 1.0)
    keep = m[...] > 0.0
    olo[...] = jnp.where(keep, slo[...] / degc, xlo[...])
    ohi[...] = jnp.where(keep, shi[...] / degc, xhi[...])


def _tc_select(sum_lo, sum_hi, deg0, deg1, maskn, x_lo, x_hi):
    bm = 1000
    grid = N // bm
    bs_h = pl.BlockSpec((bm, HALF), lambda i: (i, 0))
    bs_d = pl.BlockSpec((bm, 16), lambda i: (i, 0))
    bs_1 = pl.BlockSpec((bm, 1), lambda i: (i, 0))
    return pl.pallas_call(
        _sel_body,
        grid=(grid,),
        in_specs=[bs_h, bs_h, bs_d, bs_d, bs_1, bs_h, bs_h],
        out_specs=(bs_h, bs_h),
        out_shape=(
            jax.ShapeDtypeStruct((N, HALF), jnp.float32),
            jax.ShapeDtypeStruct((N, HALF), jnp.float32),
        ),
    )(sum_lo, sum_hi, deg0, deg1, maskn, x_lo, x_hi)


def _layer(x_lo, x_hi, ori, src2d, dst2d):
    sum_lo, sum_hi, deg0, deg1 = _sc_aggregate(x_lo, x_hi, src2d, dst2d)
    sim = _tc_sims(sum_lo, sum_hi, deg0, deg1, ori)
    sim_pad = jnp.concatenate(
        [sim.reshape(N), jnp.full((N_PAD - N,), -jnp.inf, jnp.float32)])
    mask80 = _tc_mask(sim_pad.reshape(N_PAD // 128, 128))
    maskn = mask80.reshape(N_PAD)[:N].reshape(N, 1)
    return _tc_select(sum_lo, sum_hi, deg0, deg1, maskn, x_lo, x_hi)


def kernel(features, adj_lst):
    ori = features
    x_lo = features[:, :HALF]
    x_hi = features[:, HALF:]
    for i in range(N_LAYERS):
        src2d = adj_lst[i, 0].reshape(EROWS, 128)
        dst2d = adj_lst[i, 1].reshape(EROWS, 128)
        x_lo, x_hi = _layer(x_lo, x_hi, ori, src2d, dst2d)
    return jnp.concatenate([x_lo, x_hi], axis=1)
